# baseline (speedup 1.0000x reference)
import jax
import jax.numpy as jnp
from jax.experimental import pallas as pl
from jax.experimental.pallas import tpu as pltpu

B, S, C = 32, 8192, 256
OUT_LEN = 32
RB = 2


def _stage1_body(col_ref, x_ref, conf_ref, pred_ref):
    xb = x_ref[...]
    colb = col_ref[...]
    m = jnp.max(xb, axis=-1, keepdims=True)
    s = jnp.sum(jnp.exp(xb - m), axis=-1, keepdims=True)
    p = jnp.min(jnp.where(xb == m, colb, float(C)), axis=-1, keepdims=True)
    stacked = jnp.concatenate([s, p], axis=-1)
    tr = jnp.transpose(stacked, (0, 2, 1))
    conf_ref[...] = 1.0 / tr[:, 0:1, :]
    pred_ref[...] = tr[:, 1:2, :]


def _stage2_body(conf_ref, pred_ref, out_ref):
    conf = conf_ref[...]
    pred = pred_ref[...]
    col = jax.lax.broadcasted_iota(jnp.int32, (B, S), 1)
    prev = jnp.concatenate([pred[:, :1], pred[:, :-1]], axis=1)
    boundary = (col == 0) | (pred != prev)
    t = jnp.where(boundary, col, S)
    u = jnp.concatenate([t[:, 1:], jnp.full((B, 1), S, jnp.int32)], axis=1)
    d = 1
    while d < S:
        shifted = jnp.concatenate(
            [u[:, d:], jnp.full((B, d), S, jnp.int32)], axis=1)
        u = jnp.minimum(u, shifted)
        d *= 2
    run_len = (u - col).astype(jnp.float32)
    voted = jnp.where(boundary, conf * run_len, -jnp.inf)
    outs = []
    for _ in range(OUT_LEN):
        m = jnp.max(voted, axis=1, keepdims=True)
        a = jnp.min(jnp.where(voted == m, col, S), axis=1, keepdims=True)
        onehot = col == a
        pv = jnp.max(jnp.where(onehot, pred, 0.0), axis=1, keepdims=True)
        outs.append(jnp.where(jnp.isfinite(m), pv, 0.0))
        voted = jnp.where(onehot, -jnp.inf, voted)
    out_ref[...] = jnp.concatenate(outs, axis=1)


def kernel(x):
    colc = jnp.arange(C, dtype=jnp.float32).reshape(1, 1, C)
    conf3, pred3 = pl.pallas_call(
        _stage1_body,
        grid=(B // RB,),
        in_specs=[
            pl.BlockSpec((1, 1, C), lambda b: (0, 0, 0)),
            pl.BlockSpec((RB, S, C), lambda b: (b, 0, 0)),
        ],
        out_specs=[
            pl.BlockSpec((RB, 1, S), lambda b: (b, 0, 0)),
            pl.BlockSpec((RB, 1, S), lambda b: (b, 0, 0)),
        ],
        out_shape=[
            jax.ShapeDtypeStruct((B, 1, S), jnp.float32),
            jax.ShapeDtypeStruct((B, 1, S), jnp.float32),
        ],
    )(colc, x)
    conf = conf3.reshape(B, S)
    pred = pred3.reshape(B, S)
    out = pl.pallas_call(
        _stage2_body,
        out_shape=jax.ShapeDtypeStruct((B, OUT_LEN), jnp.float32),
    )(conf, pred)
    return out.astype(x.dtype)
